# Initial kernel scaffold; baseline (speedup 1.0000x reference)
import jax
import jax.numpy as jnp
from jax.experimental import pallas as pl


def kernel(x, edge_index, edge_attr, ea_w1, ea_b1, ea_w2, ea_b2, tag1_w, tag1_b, tag2_w, tag2_b, tag3_w, tag3_b):
    raise NotImplementedError("write your pallas kernel here")


# speedup vs baseline: 1.5201x; 1.5201x over previous
import jax
import jax.numpy as jnp
from jax import lax
from jax.experimental import pallas as pl
from jax.experimental.pallas import tpu as pltpu
from jax.experimental.pallas import tpu_sc as plsc

NFEAT = 128
HID = 128
N = 10000
E = 320000
C = 128
NW = 32
CPT = 157
NCHUNK = NW * CPT
EP = NCHUNK * C
NACC = 10112
DUMMY = N
RPT = NACC // 16

MB = 1000
GRID_N = N // MB


def _sc_mesh():
    return plsc.VectorSubcoreMesh(core_axis_name="c", subcore_axis_name="s",
                                  num_cores=2, num_subcores=16)


def _zero_shared_slice(zbuf, hsh, base):
    for k in range(RPT // C):
        pltpu.sync_copy(zbuf, hsh.at[pl.ds(base + k * C, C)])
    rem = RPT % C
    if rem:
        pltpu.sync_copy(zbuf.at[pl.ds(0, rem)],
                        hsh.at[pl.ds(base + (RPT // C) * C, rem)])


def _fill_zero_buf(zbuf):
    zero16 = jnp.zeros((16,), jnp.float32)

    def zrow(i, carry):
        for g in range(HID // 16):
            zbuf[i, pl.ds(g * 16, 16)] = zero16
        return carry

    lax.fori_loop(0, C, zrow, 0)


def _msg_body(ipk, p_hbm, q_hbm, r_hbm, hacc_hbm,
              ibuf, buf, zbuf, hsh, sem):
    cid = lax.axis_index("c")
    sid = lax.axis_index("s")
    wid = cid * 16 + sid
    base = sid * RPT

    _fill_zero_buf(zbuf)
    _zero_shared_slice(zbuf, hsh, base)
    plsc.subcore_barrier()

    def chunk(c, carry):
        cidx = wid * CPT + c
        pltpu.sync_copy(ipk.at[cidx], ibuf)
        pltpu.async_copy(r_hbm.at[ibuf.at[2]], buf, sem).wait()
        d1 = pltpu.async_copy(p_hbm.at[ibuf.at[0]], buf, sem, add=True)
        d2 = pltpu.async_copy(q_hbm.at[ibuf.at[1]], buf, sem, add=True)
        d1.wait()
        d2.wait()

        def relu_row(i, rc):
            for g in range(HID // 16):
                s = pl.ds(g * 16, 16)
                buf[i, s] = jnp.maximum(buf[i, s], 0.0)
            return rc

        lax.fori_loop(0, C, relu_row, 0)
        pltpu.sync_copy(buf, hsh.at[ibuf.at[3]], add=True)
        return carry

    lax.fori_loop(0, CPT, chunk, 0)
    plsc.subcore_barrier()
    pltpu.sync_copy(hsh.at[pl.ds(base, RPT)], hacc_hbm.at[cid, pl.ds(base, RPT)])


def _prop_body(ipk, tab_hbm, acc_hbm, ibuf, buf, zbuf, hsh, sem):
    cid = lax.axis_index("c")
    sid = lax.axis_index("s")
    wid = cid * 16 + sid
    base = sid * RPT

    _fill_zero_buf(zbuf)
    _zero_shared_slice(zbuf, hsh, base)
    plsc.subcore_barrier()

    def chunk(c, carry):
        cidx = wid * CPT + c
        pltpu.sync_copy(ipk.at[cidx], ibuf)
        pltpu.async_copy(tab_hbm.at[ibuf.at[1]], buf, sem).wait()
        pltpu.sync_copy(buf, hsh.at[ibuf.at[3]], add=True)
        return carry

    lax.fori_loop(0, CPT, chunk, 0)
    plsc.subcore_barrier()
    pltpu.sync_copy(hsh.at[pl.ds(base, RPT)], acc_hbm.at[cid, pl.ds(base, RPT)])


def _msg_call(ipk, P, Q, R):
    return pl.kernel(
        _msg_body,
        out_type=jax.ShapeDtypeStruct((2, NACC, HID), jnp.float32),
        mesh=_sc_mesh(),
        scratch_types=[
            pltpu.VMEM((4, C), jnp.int32),
            pltpu.VMEM((C, HID), jnp.float32),
            pltpu.VMEM((C, HID), jnp.float32),
            pltpu.VMEM_SHARED((NACC, HID), jnp.float32),
            pltpu.SemaphoreType.DMA,
        ],
    )(ipk, P, Q, R)


def _prop_call(ipk, table):
    return pl.kernel(
        _prop_body,
        out_type=jax.ShapeDtypeStruct((2, NACC, HID), jnp.float32),
        mesh=_sc_mesh(),
        scratch_types=[
            pltpu.VMEM((4, C), jnp.int32),
            pltpu.VMEM((C, HID), jnp.float32),
            pltpu.VMEM((C, HID), jnp.float32),
            pltpu.VMEM_SHARED((NACC, HID), jnp.float32),
            pltpu.SemaphoreType.DMA,
        ],
    )(ipk, table)


def _pre1_body(xf_ref, w_ref, o_ref):
    o_ref[...] = jnp.dot(xf_ref[...], w_ref[...],
                         preferred_element_type=jnp.float32)


def _pre1(xf, w01):
    return pl.pallas_call(
        _pre1_body,
        grid=(GRID_N,),
        in_specs=[pl.BlockSpec((MB, NFEAT), lambda i: (i, 0)),
                  pl.BlockSpec((NFEAT, 2 * HID), lambda i: (0, 0))],
        out_specs=pl.BlockSpec((MB, 2 * HID), lambda i: (i, 0)),
        out_shape=jax.ShapeDtypeStruct((N, 2 * HID), jnp.float32),
    )(xf, w01)


def _pre2_body(ea_ref, w_ref, b_ref, r_ref):
    r_ref[...] = (jnp.dot(ea_ref[...], w_ref[...],
                          preferred_element_type=jnp.float32) + b_ref[...])


def _pre2(ea, w2, b1):
    EB = 8000
    return pl.pallas_call(
        _pre2_body,
        grid=(E // EB,),
        in_specs=[pl.BlockSpec((EB, 16), lambda i: (i, 0)),
                  pl.BlockSpec((16, HID), lambda i: (0, 0)),
                  pl.BlockSpec((1, HID), lambda i: (0, 0))],
        out_specs=pl.BlockSpec((EB, HID), lambda i: (i, 0)),
        out_shape=jax.ShapeDtypeStruct((E, HID), jnp.float32),
    )(ea, w2, b1)


def _combine_body(hacc_ref, deg_ref, w2_ref, b2_ref, h_ref, t1_ref, dist_ref):
    hs = hacc_ref[0] + hacc_ref[1]
    deg = (deg_ref[0, :, :1] + deg_ref[1, :, :1])
    h = jnp.dot(hs, w2_ref[...], preferred_element_type=jnp.float32) \
        + deg * b2_ref[...]
    dist = jnp.where(deg > 0, lax.rsqrt(deg), 0.0)
    h_ref[...] = h
    t1_ref[...] = dist * h
    dist_ref[...] = dist


def _combine(hacc, degp, w2, b2):
    return pl.pallas_call(
        _combine_body,
        grid=(GRID_N,),
        in_specs=[pl.BlockSpec((2, MB, HID), lambda i: (0, i, 0)),
                  pl.BlockSpec((2, MB, HID), lambda i: (0, i, 0)),
                  pl.BlockSpec((HID, HID), lambda i: (0, 0)),
                  pl.BlockSpec((1, HID), lambda i: (0, 0))],
        out_specs=[pl.BlockSpec((MB, HID), lambda i: (i, 0)),
                   pl.BlockSpec((MB, HID), lambda i: (i, 0)),
                   pl.BlockSpec((MB, 1), lambda i: (i, 0))],
        out_shape=[jax.ShapeDtypeStruct((N, HID), jnp.float32),
                   jax.ShapeDtypeStruct((N, HID), jnp.float32),
                   jax.ShapeDtypeStruct((N, 1), jnp.float32)],
    )(hacc, degp, w2, b2)


def _mid_body(u_ref, dist_ref, su_ref, t2_ref):
    u = u_ref[0] + u_ref[1]
    dist = dist_ref[...]
    su = dist * u
    su_ref[...] = su
    t2_ref[...] = dist * su


def _mid(uacc, dist):
    return pl.pallas_call(
        _mid_body,
        grid=(GRID_N,),
        in_specs=[pl.BlockSpec((2, MB, HID), lambda i: (0, i, 0)),
                  pl.BlockSpec((MB, 1), lambda i: (i, 0))],
        out_specs=[pl.BlockSpec((MB, HID), lambda i: (i, 0)),
                   pl.BlockSpec((MB, HID), lambda i: (i, 0))],
        out_shape=[jax.ShapeDtypeStruct((N, HID), jnp.float32),
                   jax.ShapeDtypeStruct((N, HID), jnp.float32)],
    )(uacc, dist)


def _post_relu_body(h_ref, su_ref, v_ref, dist_ref, w_ref, b_ref,
                    hn_ref, tn_ref):
    dist = dist_ref[...]
    sv = dist * (v_ref[0] + v_ref[1])
    out = (jnp.dot(h_ref[...], w_ref[0], preferred_element_type=jnp.float32)
           + jnp.dot(su_ref[...], w_ref[1], preferred_element_type=jnp.float32)
           + jnp.dot(sv, w_ref[2], preferred_element_type=jnp.float32)
           + b_ref[...])
    hn = jnp.maximum(out, 0.0)
    hn_ref[...] = hn
    tn_ref[...] = dist * hn


def _post_final_body(h_ref, su_ref, v_ref, dist_ref, w_ref, b_ref, out_ref):
    dist = dist_ref[...]
    sv = dist * (v_ref[0] + v_ref[1])
    out_ref[...] = (
        jnp.dot(h_ref[...], w_ref[0], preferred_element_type=jnp.float32)
        + jnp.dot(su_ref[...], w_ref[1], preferred_element_type=jnp.float32)
        + jnp.dot(sv, w_ref[2], preferred_element_type=jnp.float32)
        + b_ref[...])


def _post(h, su, vacc, dist, w, b, final):
    in_specs = [pl.BlockSpec((MB, HID), lambda i: (i, 0)),
                pl.BlockSpec((MB, HID), lambda i: (i, 0)),
                pl.BlockSpec((2, MB, HID), lambda i: (0, i, 0)),
                pl.BlockSpec((MB, 1), lambda i: (i, 0)),
                pl.BlockSpec((3, HID, HID), lambda i: (0, 0, 0)),
                pl.BlockSpec((1, HID), lambda i: (0, 0))]
    if final:
        return pl.pallas_call(
            _post_final_body,
            grid=(GRID_N,),
            in_specs=in_specs,
            out_specs=pl.BlockSpec((MB, HID), lambda i: (i, 0)),
            out_shape=jax.ShapeDtypeStruct((N, HID), jnp.float32),
        )(h, su, vacc, dist, w, b)
    return pl.pallas_call(
        _post_relu_body,
        grid=(GRID_N,),
        in_specs=in_specs,
        out_specs=[pl.BlockSpec((MB, HID), lambda i: (i, 0)),
                   pl.BlockSpec((MB, HID), lambda i: (i, 0))],
        out_shape=[jax.ShapeDtypeStruct((N, HID), jnp.float32),
                   jax.ShapeDtypeStruct((N, HID), jnp.float32)],
    )(h, su, vacc, dist, w, b)


def kernel(x, edge_index, edge_attr, ea_w1, ea_b1, ea_w2, ea_b2,
           tag1_w, tag1_b, tag2_w, tag2_b, tag3_w, tag3_b):
    xf = x[:, 4:4 + NFEAT]
    ei = edge_index.astype(jnp.int32)
    row0, col0 = ei[0], ei[1]
    directed = ~jnp.any((row0 == col0[0]) & (col0 == row0[0]))

    pad = EP - 2 * E
    zpad = jnp.zeros((pad,), jnp.int32)
    ar = jnp.arange(E, dtype=jnp.int32)
    gcol = jnp.concatenate([col0, row0, zpad])
    grow = jnp.concatenate([row0, col0, zpad])
    ridx = jnp.concatenate([ar, ar, zpad])
    scat = jnp.concatenate([col0, jnp.where(directed, row0, DUMMY),
                            jnp.full((pad,), DUMMY, jnp.int32)])
    ipk = jnp.stack([gcol, grow, ridx, scat], axis=0)
    ipk = ipk.reshape(4, NCHUNK, C).transpose(1, 0, 2)
    ipk_deg = ipk.at[:, 1, :].set(0)

    w01 = jnp.concatenate([ea_w1[:NFEAT], ea_w1[NFEAT:2 * NFEAT]], axis=1)
    PQ = _pre1(xf, w01)
    P = PQ[:, :HID]
    Q = PQ[:, HID:]
    R = _pre2(edge_attr, ea_w1[2 * NFEAT:], ea_b1.reshape(1, HID))

    hacc = _msg_call(ipk, P, Q, R)
    degp = _prop_call(ipk_deg, jnp.ones((N, HID), jnp.float32))
    h, t, dist = _combine(hacc, degp, ea_w2, ea_b2.reshape(1, HID))

    for (w, b, final) in ((tag1_w, tag1_b, False),
                          (tag2_w, tag2_b, False),
                          (tag3_w, tag3_b, True)):
        uacc = _prop_call(ipk, t)
        su, t2 = _mid(uacc, dist)
        vacc = _prop_call(ipk, t2)
        res = _post(h, su, vacc, dist, w, b.reshape(1, HID), final)
        if final:
            return res
        h, t = res


# speedup vs baseline: 8.4080x; 5.5312x over previous
import jax
import jax.numpy as jnp
from jax import lax
from jax.experimental import pallas as pl
from jax.experimental.pallas import tpu as pltpu
from jax.experimental.pallas import tpu_sc as plsc

NFEAT = 128
HID = 128
N = 10000
E = 320000
C = 128
NW = 32
CPT = 157
NCHUNK = NW * CPT
EP = NCHUNK * C
CM = 64
CPTM = EP // (NW * CM)
NCHUNKM = NW * CPTM
NACC = 10112
DUMMY = N
RPT = NACC // 16

MB = 1000
GRID_N = N // MB


def _sc_mesh():
    return plsc.VectorSubcoreMesh(core_axis_name="c", subcore_axis_name="s",
                                  num_cores=2, num_subcores=16)


def _zero_shared_slice(zbuf, hsh, base):
    for k in range(RPT // C):
        pltpu.sync_copy(zbuf, hsh.at[pl.ds(base + k * C, C)])
    rem = RPT % C
    if rem:
        pltpu.sync_copy(zbuf.at[pl.ds(0, rem)],
                        hsh.at[pl.ds(base + (RPT // C) * C, rem)])


def _fill_zero_buf(zbuf):
    zero16 = jnp.zeros((16,), jnp.float32)

    def zrow(i, carry):
        for g in range(HID // 16):
            zbuf[i, pl.ds(g * 16, 16)] = zero16
        return carry

    lax.fori_loop(0, C, zrow, 0)


def _msg_body(ipk, p_hbm, q_hbm, r_hbm, hacc_hbm,
              ibuf, buf, bufp, bufq, hsh, sem):
    cid = lax.axis_index("c")
    sid = lax.axis_index("s")
    wid = cid * 16 + sid
    base = sid * RPT

    zero16 = jnp.zeros((16,), jnp.float32)

    def zrow(i, carry):
        for g in range(HID // 16):
            bufp[i, pl.ds(g * 16, 16)] = zero16
        return carry

    lax.fori_loop(0, CM, zrow, 0)
    for k in range(RPT // CM):
        pltpu.sync_copy(bufp, hsh.at[pl.ds(base + k * CM, CM)])
    rem = RPT % CM
    if rem:
        pltpu.sync_copy(bufp.at[pl.ds(0, rem)],
                        hsh.at[pl.ds(base + (RPT // CM) * CM, rem)])
    plsc.subcore_barrier()

    def chunk(c, carry):
        cidx = wid * CPTM + c
        pltpu.sync_copy(ipk.at[cidx], ibuf)
        rbase = lax.rem(cidx * CM, E)
        d0 = pltpu.async_copy(r_hbm.at[pl.ds(rbase, CM)], buf, sem)
        d1 = pltpu.async_copy(p_hbm.at[ibuf.at[0]], bufp, sem)
        d2 = pltpu.async_copy(q_hbm.at[ibuf.at[1]], bufq, sem)
        d0.wait()
        d1.wait()
        d2.wait()

        def relu_row(i, rc):
            for g in range(HID // 16):
                s = pl.ds(g * 16, 16)
                buf[i, s] = jnp.maximum(buf[i, s] + bufp[i, s] + bufq[i, s],
                                        0.0)
            return rc

        lax.fori_loop(0, CM, relu_row, 0)
        pltpu.sync_copy(buf, hsh.at[ibuf.at[3]], add=True)
        return carry

    lax.fori_loop(0, CPTM, chunk, 0)
    plsc.subcore_barrier()
    pltpu.sync_copy(hsh.at[pl.ds(base, RPT)], hacc_hbm.at[cid, pl.ds(base, RPT)])


def _prop_body(ipk, tab_hbm, acc_hbm, ibuf, buf, zbuf, hsh, sem):
    cid = lax.axis_index("c")
    sid = lax.axis_index("s")
    wid = cid * 16 + sid
    base = sid * RPT

    _fill_zero_buf(zbuf)
    _zero_shared_slice(zbuf, hsh, base)
    plsc.subcore_barrier()

    def chunk(c, carry):
        cidx = wid * CPT + c
        pltpu.sync_copy(ipk.at[cidx], ibuf)
        pltpu.async_copy(tab_hbm.at[ibuf.at[1]], buf, sem).wait()
        pltpu.sync_copy(buf, hsh.at[ibuf.at[3]], add=True)
        return carry

    lax.fori_loop(0, CPT, chunk, 0)
    plsc.subcore_barrier()
    pltpu.sync_copy(hsh.at[pl.ds(base, RPT)], acc_hbm.at[cid, pl.ds(base, RPT)])


def _msg_call(ipk, P, Q, R):
    return pl.kernel(
        _msg_body,
        out_type=jax.ShapeDtypeStruct((2, NACC, HID), jnp.float32),
        mesh=_sc_mesh(),
        scratch_types=[
            pltpu.VMEM((4, CM), jnp.int32),
            pltpu.VMEM((CM, HID), jnp.float32),
            pltpu.VMEM((CM, HID), jnp.float32),
            pltpu.VMEM((CM, HID), jnp.float32),
            pltpu.VMEM_SHARED((NACC, HID), jnp.float32),
            pltpu.SemaphoreType.DMA,
        ],
    )(ipk, P, Q, R)


def _prop_call(ipk, table):
    return pl.kernel(
        _prop_body,
        out_type=jax.ShapeDtypeStruct((2, NACC, HID), jnp.float32),
        mesh=_sc_mesh(),
        scratch_types=[
            pltpu.VMEM((4, C), jnp.int32),
            pltpu.VMEM((C, HID), jnp.float32),
            pltpu.VMEM((C, HID), jnp.float32),
            pltpu.VMEM_SHARED((NACC, HID), jnp.float32),
            pltpu.SemaphoreType.DMA,
        ],
    )(ipk, table)


def _pre1_body(xf_ref, w_ref, o_ref):
    o_ref[...] = jnp.dot(xf_ref[...], w_ref[...],
                         preferred_element_type=jnp.float32)


def _pre1(xf, w01):
    return pl.pallas_call(
        _pre1_body,
        grid=(GRID_N,),
        in_specs=[pl.BlockSpec((MB, NFEAT), lambda i: (i, 0)),
                  pl.BlockSpec((NFEAT, 2 * HID), lambda i: (0, 0))],
        out_specs=pl.BlockSpec((MB, 2 * HID), lambda i: (i, 0)),
        out_shape=jax.ShapeDtypeStruct((N, 2 * HID), jnp.float32),
    )(xf, w01)


def _pre2_body(ea_ref, w_ref, b_ref, r_ref):
    r_ref[...] = (jnp.dot(ea_ref[...], w_ref[...],
                          preferred_element_type=jnp.float32) + b_ref[...])


def _pre2(ea, w2, b1):
    EB = 8000
    return pl.pallas_call(
        _pre2_body,
        grid=(E // EB,),
        in_specs=[pl.BlockSpec((EB, 16), lambda i: (i, 0)),
                  pl.BlockSpec((16, HID), lambda i: (0, 0)),
                  pl.BlockSpec((1, HID), lambda i: (0, 0))],
        out_specs=pl.BlockSpec((EB, HID), lambda i: (i, 0)),
        out_shape=jax.ShapeDtypeStruct((E, HID), jnp.float32),
    )(ea, w2, b1)


def _combine_body(hacc_ref, deg_ref, w2_ref, b2_ref, h_ref, t1_ref, dist_ref):
    hs = hacc_ref[0] + hacc_ref[1]
    deg = (deg_ref[0, :, :1] + deg_ref[1, :, :1])
    h = jnp.dot(hs, w2_ref[...], preferred_element_type=jnp.float32) \
        + deg * b2_ref[...]
    dist = jnp.where(deg > 0, lax.rsqrt(deg), 0.0)
    h_ref[...] = h
    t1_ref[...] = dist * h
    dist_ref[...] = dist


def _combine(hacc, degp, w2, b2):
    return pl.pallas_call(
        _combine_body,
        grid=(GRID_N,),
        in_specs=[pl.BlockSpec((2, MB, HID), lambda i: (0, i, 0)),
                  pl.BlockSpec((2, MB, HID), lambda i: (0, i, 0)),
                  pl.BlockSpec((HID, HID), lambda i: (0, 0)),
                  pl.BlockSpec((1, HID), lambda i: (0, 0))],
        out_specs=[pl.BlockSpec((MB, HID), lambda i: (i, 0)),
                   pl.BlockSpec((MB, HID), lambda i: (i, 0)),
                   pl.BlockSpec((MB, 1), lambda i: (i, 0))],
        out_shape=[jax.ShapeDtypeStruct((N, HID), jnp.float32),
                   jax.ShapeDtypeStruct((N, HID), jnp.float32),
                   jax.ShapeDtypeStruct((N, 1), jnp.float32)],
    )(hacc, degp, w2, b2)


def _mid_body(u_ref, dist_ref, su_ref, t2_ref):
    u = u_ref[0] + u_ref[1]
    dist = dist_ref[...]
    su = dist * u
    su_ref[...] = su
    t2_ref[...] = dist * su


def _mid(uacc, dist):
    return pl.pallas_call(
        _mid_body,
        grid=(GRID_N,),
        in_specs=[pl.BlockSpec((2, MB, HID), lambda i: (0, i, 0)),
                  pl.BlockSpec((MB, 1), lambda i: (i, 0))],
        out_specs=[pl.BlockSpec((MB, HID), lambda i: (i, 0)),
                   pl.BlockSpec((MB, HID), lambda i: (i, 0))],
        out_shape=[jax.ShapeDtypeStruct((N, HID), jnp.float32),
                   jax.ShapeDtypeStruct((N, HID), jnp.float32)],
    )(uacc, dist)


def _post_relu_body(h_ref, su_ref, v_ref, dist_ref, w_ref, b_ref,
                    hn_ref, tn_ref):
    dist = dist_ref[...]
    sv = dist * (v_ref[0] + v_ref[1])
    out = (jnp.dot(h_ref[...], w_ref[0], preferred_element_type=jnp.float32)
           + jnp.dot(su_ref[...], w_ref[1], preferred_element_type=jnp.float32)
           + jnp.dot(sv, w_ref[2], preferred_element_type=jnp.float32)
           + b_ref[...])
    hn = jnp.maximum(out, 0.0)
    hn_ref[...] = hn
    tn_ref[...] = dist * hn


def _post_final_body(h_ref, su_ref, v_ref, dist_ref, w_ref, b_ref, out_ref):
    dist = dist_ref[...]
    sv = dist * (v_ref[0] + v_ref[1])
    out_ref[...] = (
        jnp.dot(h_ref[...], w_ref[0], preferred_element_type=jnp.float32)
        + jnp.dot(su_ref[...], w_ref[1], preferred_element_type=jnp.float32)
        + jnp.dot(sv, w_ref[2], preferred_element_type=jnp.float32)
        + b_ref[...])


def _post(h, su, vacc, dist, w, b, final):
    in_specs = [pl.BlockSpec((MB, HID), lambda i: (i, 0)),
                pl.BlockSpec((MB, HID), lambda i: (i, 0)),
                pl.BlockSpec((2, MB, HID), lambda i: (0, i, 0)),
                pl.BlockSpec((MB, 1), lambda i: (i, 0)),
                pl.BlockSpec((3, HID, HID), lambda i: (0, 0, 0)),
                pl.BlockSpec((1, HID), lambda i: (0, 0))]
    if final:
        return pl.pallas_call(
            _post_final_body,
            grid=(GRID_N,),
            in_specs=in_specs,
            out_specs=pl.BlockSpec((MB, HID), lambda i: (i, 0)),
            out_shape=jax.ShapeDtypeStruct((N, HID), jnp.float32),
        )(h, su, vacc, dist, w, b)
    return pl.pallas_call(
        _post_relu_body,
        grid=(GRID_N,),
        in_specs=in_specs,
        out_specs=[pl.BlockSpec((MB, HID), lambda i: (i, 0)),
                   pl.BlockSpec((MB, HID), lambda i: (i, 0))],
        out_shape=[jax.ShapeDtypeStruct((N, HID), jnp.float32),
                   jax.ShapeDtypeStruct((N, HID), jnp.float32)],
    )(h, su, vacc, dist, w, b)


def kernel(x, edge_index, edge_attr, ea_w1, ea_b1, ea_w2, ea_b2,
           tag1_w, tag1_b, tag2_w, tag2_b, tag3_w, tag3_b):
    xf = x[:, 4:4 + NFEAT]
    ei = edge_index.astype(jnp.int32)
    row0, col0 = ei[0], ei[1]
    directed = ~jnp.any((row0 == col0[0]) & (col0 == row0[0]))

    pad = EP - 2 * E
    zpad = jnp.zeros((pad,), jnp.int32)
    ar = jnp.arange(E, dtype=jnp.int32)
    gcol = jnp.concatenate([col0, row0, zpad])
    grow = jnp.concatenate([row0, col0, zpad])
    ridx = jnp.concatenate([ar, ar, zpad])
    scat = jnp.concatenate([col0, jnp.where(directed, row0, DUMMY),
                            jnp.full((pad,), DUMMY, jnp.int32)])
    packed = jnp.stack([gcol, grow, ridx, scat], axis=0)
    ipk = packed.reshape(4, NCHUNK, C).transpose(1, 0, 2)
    ipkm = packed.reshape(4, NCHUNKM, CM).transpose(1, 0, 2)
    ipk_deg = jnp.stack([gcol, gcol, ridx, scat], axis=0)
    ipk_deg = ipk_deg.reshape(4, NCHUNK, C).transpose(1, 0, 2)

    w01 = jnp.concatenate([ea_w1[:NFEAT], ea_w1[NFEAT:2 * NFEAT]], axis=1)
    PQ = _pre1(xf, w01)
    P = PQ[:, :HID]
    Q = PQ[:, HID:]
    R = _pre2(edge_attr, ea_w1[2 * NFEAT:], ea_b1.reshape(1, HID))

    hacc = _msg_call(ipkm, P, Q, R)
    degp = _prop_call(ipk_deg, jnp.ones((N, HID), jnp.float32))
    h, t, dist = _combine(hacc, degp, ea_w2, ea_b2.reshape(1, HID))

    for (w, b, final) in ((tag1_w, tag1_b, False),
                          (tag2_w, tag2_b, False),
                          (tag3_w, tag3_b, True)):
        uacc = _prop_call(ipk, t)
        su, t2 = _mid(uacc, dist)
        vacc = _prop_call(ipk, t2)
        res = _post(h, su, vacc, dist, w, b.reshape(1, HID), final)
        if final:
            return res
        h, t = res
